# baseline (speedup 1.0000x reference)
import jax
import jax.numpy as jnp
from jax import lax
from jax.experimental import pallas as pl
from jax.experimental.pallas import tpu as pltpu
from jax.experimental.pallas import tpu_sc as plsc

N = 10000
D = 128
L = 320000
T = L // 2

K_SHIFT = 44.0
LOG_EPS = -3.0701134573253944

_NS = 16
_NW = 32
_WR = L // _NW

_GCH = 200
_GSUBS = ((0, 104), (104, 96))
_GNCH = _WR // _GCH

_SCH = 80
_SNCH = _WR // _SCH
N_PAD = 10240
_NODES_PER_TILE = N_PAD // _NS

_REL_BLK = 1000
_FIN_BLK = 1000


def _sc_mesh():
    return plsc.VectorSubcoreMesh(core_axis_name="c", subcore_axis_name="s")


def _gather_body(emb_hbm, idx_hbm, x_hbm, idx_v, buf_a, buf_b, ga, gb, sa, sb):
    c = lax.axis_index("c")
    s = lax.axis_index("s")
    w = s * 2 + c
    base = pl.multiple_of(w * _WR, _WR)
    pltpu.sync_copy(idx_hbm.at[pl.ds(base, _WR)], idx_v)

    def start_gather(chunk, buf, sem):
        off = chunk * _GCH
        for q, sz in _GSUBS:
            pltpu.async_copy(
                emb_hbm.at[idx_v.at[pl.ds(off + q, sz)]],
                buf.at[pl.ds(q, sz)], sem)

    def wait_gather(chunk, buf, sem):
        off = chunk * _GCH
        for q, sz in _GSUBS:
            pltpu.make_async_copy(
                emb_hbm.at[idx_v.at[pl.ds(off + q, sz)]],
                buf.at[pl.ds(q, sz)], sem).wait()

    def x_dst(chunk):
        return x_hbm.at[pl.ds(base + chunk * _GCH, _GCH)]

    start_gather(0, buf_a, ga)

    def pair(k, carry):
        c0 = 2 * k
        c1 = c0 + 1

        @pl.when(k > 0)
        def _():
            pltpu.make_async_copy(buf_b, x_dst(c0 - 1), sb).wait()

        start_gather(c1, buf_b, gb)
        wait_gather(c0, buf_a, ga)
        pltpu.async_copy(buf_a, x_dst(c0), sa)
        wait_gather(c1, buf_b, gb)
        pltpu.async_copy(buf_b, x_dst(c1), sb)
        pltpu.make_async_copy(buf_a, x_dst(c0), sa).wait()

        @pl.when(k < _GNCH // 2 - 1)
        def _():
            start_gather(c0 + 2, buf_a, ga)

        return carry

    lax.fori_loop(0, _GNCH // 2, pair, 0)
    pltpu.make_async_copy(buf_b, x_dst(_GNCH - 1), sb).wait()


def _gather_sc(emb, idx):
    return pl.kernel(
        _gather_body,
        out_type=jax.ShapeDtypeStruct((L, D), jnp.float32),
        mesh=_sc_mesh(),
        scratch_types=[
            pltpu.VMEM((_WR,), jnp.int32),
            pltpu.VMEM((_GCH, D), jnp.float32),
            pltpu.VMEM((_GCH, D), jnp.float32),
            pltpu.SemaphoreType.DMA,
            pltpu.SemaphoreType.DMA,
            pltpu.SemaphoreType.DMA,
            pltpu.SemaphoreType.DMA,
        ],
    )(emb, idx)


def _scatter_body(z_hbm, idx_hbm, s_hbm,
                  idx_v, buf_a, buf_b, acc, ia, la, lb, wa, wb):
    c = lax.axis_index("c")
    s = lax.axis_index("s")
    w = s * 2 + c
    base = pl.multiple_of(w * _WR, _WR)
    nbase = s * _NODES_PER_TILE

    zvec = jnp.zeros((16,), jnp.float32)
    for row in range(40):
        for q in range(D // 16):
            buf_a[row, pl.ds(16 * q, 16)] = zvec
    for q in range(_NODES_PER_TILE // 40):
        pltpu.sync_copy(buf_a.at[pl.ds(0, 40)],
                        acc.at[pl.ds(nbase + q * 40, 40)])
    plsc.subcore_barrier()

    def idx_src(j):
        return idx_hbm.at[pl.ds(base + j * _SCH, _SCH)]

    def fire(j, carry):
        pltpu.async_copy(idx_src(j), idx_v.at[j], ia)
        return carry
    lax.fori_loop(0, _SNCH, fire, 0)

    def drain(j, carry):
        pltpu.make_async_copy(idx_src(j), idx_v.at[j], ia).wait()
        return carry
    lax.fori_loop(0, _SNCH, drain, 0)

    def z_src(j):
        return z_hbm.at[pl.ds(base + j * _SCH, _SCH)]

    def start_scatter(j, buf, sem):
        pltpu.async_copy(buf, acc.at[idx_v.at[j]], sem, add=True)

    def wait_scatter(j, buf, sem):
        pltpu.make_async_copy(buf, acc.at[idx_v.at[j]], sem).wait()

    pltpu.async_copy(z_src(0), buf_a, la)

    def pair(k, carry2):
        j0 = 2 * k
        j1 = j0 + 1

        @pl.when(k > 0)
        def _():
            wait_scatter(j0 - 1, buf_b, wb)

        pltpu.async_copy(z_src(j1), buf_b, lb)
        pltpu.make_async_copy(z_src(j0), buf_a, la).wait()
        start_scatter(j0, buf_a, wa)
        pltpu.make_async_copy(z_src(j1), buf_b, lb).wait()
        start_scatter(j1, buf_b, wb)
        wait_scatter(j0, buf_a, wa)

        @pl.when(k < _SNCH // 2 - 1)
        def _():
            pltpu.async_copy(z_src(j0 + 2), buf_a, la)

        return carry2

    lax.fori_loop(0, _SNCH // 2, pair, 0)

    tail = _SNCH - 1
    pltpu.async_copy(z_src(tail), buf_a, la)
    pltpu.make_async_copy(z_src(tail), buf_a, la).wait()
    start_scatter(tail, buf_a, wa)
    wait_scatter(tail - 1, buf_b, wb)
    wait_scatter(tail, buf_a, wa)

    plsc.subcore_barrier()
    pltpu.sync_copy(acc.at[pl.ds(nbase, _NODES_PER_TILE)],
                    s_hbm.at[c, pl.ds(nbase, _NODES_PER_TILE)])


def _scatter_sc(z, idx):
    return pl.kernel(
        _scatter_body,
        out_type=jax.ShapeDtypeStruct((2, N_PAD, D), jnp.float32),
        mesh=_sc_mesh(),
        scratch_types=[
            pltpu.VMEM((_SNCH, _SCH), jnp.int32),
            pltpu.VMEM((_SCH, D), jnp.float32),
            pltpu.VMEM((_SCH, D), jnp.float32),
            pltpu.VMEM_SHARED((N_PAD, D), jnp.float32),
            pltpu.SemaphoreType.DMA,
            pltpu.SemaphoreType.DMA,
            pltpu.SemaphoreType.DMA,
            pltpu.SemaphoreType.DMA,
            pltpu.SemaphoreType.DMA,
        ],
    )(z, idx)


def _mish(t):
    w = jnp.exp(jnp.minimum(t, 30.0))
    n = w * (w + 2.0)
    return t * (n / (n + 2.0))


def _rel_body(x_ref, wrt_ref, br_ref, wot_ref, bo_ref, z_ref):
    x = x_ref[...].reshape(_REL_BLK, 2 * D)
    t = jnp.dot(x.astype(jnp.bfloat16), wrt_ref[...],
                preferred_element_type=jnp.float32) + br_ref[...]
    h = x + _mish(t)
    e = jnp.dot(h.astype(jnp.bfloat16), wot_ref[...],
                preferred_element_type=jnp.float32) + bo_ref[...]
    z_ref[...] = jnp.exp(e).reshape(2 * _REL_BLK, D)


def _relation_z(x, WrT, br2, WoT, bo2):
    grid = T // _REL_BLK
    return pl.pallas_call(
        _rel_body,
        grid=(grid,),
        in_specs=[
            pl.BlockSpec((2 * _REL_BLK, D), lambda i: (i, 0)),
            pl.BlockSpec((2 * D, 2 * D), lambda i: (0, 0)),
            pl.BlockSpec((1, 2 * D), lambda i: (0, 0)),
            pl.BlockSpec((2 * D, 2 * D), lambda i: (0, 0)),
            pl.BlockSpec((1, 2 * D), lambda i: (0, 0)),
        ],
        out_specs=pl.BlockSpec((2 * _REL_BLK, D), lambda i: (i, 0)),
        out_shape=jax.ShapeDtypeStruct((L, D), jnp.float32),
    )(x, WrT, br2, WoT, bo2)


def _final_body(s0_ref, s1_ref, emb_ref, wrut_ref, bru_ref, wout_ref,
                bou_ref, o_ref):
    sacc = (s0_ref[0] + s0_ref[1]) + (s1_ref[0] + s1_ref[1])
    emb = emb_ref[...]
    max_msg = jnp.where(sacc == 0.0, LOG_EPS,
                        (jnp.log(sacc) + K_SHIFT) * (1.0 / 12.0))
    x = jnp.concatenate([max_msg, emb], axis=1)
    t = jnp.dot(x, wrut_ref[...], preferred_element_type=jnp.float32) + bru_ref[...]
    h = x + _mish(t)
    o_ref[...] = jnp.dot(h, wout_ref[...], preferred_element_type=jnp.float32) + bou_ref[...]


def _final_update(s0, s1, emb, WruT, bru2, WouT, bou2):
    grid = N // _FIN_BLK
    return pl.pallas_call(
        _final_body,
        grid=(grid,),
        in_specs=[
            pl.BlockSpec((2, _FIN_BLK, D), lambda i: (0, i, 0)),
            pl.BlockSpec((2, _FIN_BLK, D), lambda i: (0, i, 0)),
            pl.BlockSpec((_FIN_BLK, D), lambda i: (i, 0)),
            pl.BlockSpec((2 * D, 2 * D), lambda i: (0, 0)),
            pl.BlockSpec((1, 2 * D), lambda i: (0, 0)),
            pl.BlockSpec((2 * D, D), lambda i: (0, 0)),
            pl.BlockSpec((1, D), lambda i: (0, 0)),
        ],
        out_specs=pl.BlockSpec((_FIN_BLK, D), lambda i: (i, 0)),
        out_shape=jax.ShapeDtypeStruct((N, D), jnp.float32),
    )(s0, s1, emb, WruT, bru2, WouT, bou2)


def kernel(object_embeddings, rel0_values, rel1_values,
           Wr0, br0, Wo0, bo0, Wr1, br1, Wo1, bo1,
           Wru, bru, Wou, bou):
    emb = object_embeddings

    bf = jnp.bfloat16
    x0 = _gather_sc(emb, rel0_values)
    z0 = _relation_z(x0, Wr0.T.astype(bf), br0.reshape(1, -1),
                     (12.0 * Wo0.T).astype(bf),
                     (12.0 * bo0 - K_SHIFT).reshape(1, -1))
    s0 = _scatter_sc(z0, rel0_values)

    x1 = _gather_sc(emb, rel1_values)
    z1 = _relation_z(x1, Wr1.T.astype(bf), br1.reshape(1, -1),
                     (12.0 * Wo1.T).astype(bf),
                     (12.0 * bo1 - K_SHIFT).reshape(1, -1))
    s1 = _scatter_sc(z1, rel1_values)

    return _final_update(s0, s1, emb, Wru.T, bru.reshape(1, -1),
                         Wou.T, bou.reshape(1, -1))


# speedup vs baseline: 1.0858x; 1.0858x over previous
import jax
import jax.numpy as jnp
from jax import lax
from jax.experimental import pallas as pl
from jax.experimental.pallas import tpu as pltpu
from jax.experimental.pallas import tpu_sc as plsc

N = 10000
D = 128
L = 320000
T = L // 2

K_SHIFT = 44.0
LOG_EPS = -3.0701134573253944

_NS = 16
_NW = 32
_WR = L // _NW

_GCH = 200
_GSUBS = ((0, 104), (104, 96))
_GNCH = _WR // _GCH

_SCH = 80
_SNCH = _WR // _SCH
N_PAD = 10240
_NODES_PER_TILE = N_PAD // _NS

_REL_BLK = 2000
_FIN_BLK = 1000


def _sc_mesh():
    return plsc.VectorSubcoreMesh(core_axis_name="c", subcore_axis_name="s")


def _gather_body(emb_hbm, idx_hbm, x_hbm, idx_v, buf_a, buf_b, ga, gb, sa, sb):
    c = lax.axis_index("c")
    s = lax.axis_index("s")
    w = s * 2 + c
    base = pl.multiple_of(w * _WR, _WR)
    pltpu.sync_copy(idx_hbm.at[pl.ds(base, _WR)], idx_v)

    def start_gather(chunk, buf, sem):
        off = chunk * _GCH
        for q, sz in _GSUBS:
            pltpu.async_copy(
                emb_hbm.at[idx_v.at[pl.ds(off + q, sz)]],
                buf.at[pl.ds(q, sz)], sem)

    def wait_gather(chunk, buf, sem):
        off = chunk * _GCH
        for q, sz in _GSUBS:
            pltpu.make_async_copy(
                emb_hbm.at[idx_v.at[pl.ds(off + q, sz)]],
                buf.at[pl.ds(q, sz)], sem).wait()

    def x_dst(chunk):
        return x_hbm.at[pl.ds(base + chunk * _GCH, _GCH)]

    start_gather(0, buf_a, ga)

    def pair(k, carry):
        c0 = 2 * k
        c1 = c0 + 1
        wait_gather(c0, buf_a, ga)
        pltpu.async_copy(buf_a, x_dst(c0), sa)

        @pl.when(k > 0)
        def _():
            pltpu.make_async_copy(buf_b, x_dst(c0 - 1), sb).wait()

        start_gather(c1, buf_b, gb)
        wait_gather(c1, buf_b, gb)
        pltpu.async_copy(buf_b, x_dst(c1), sb)
        pltpu.make_async_copy(buf_a, x_dst(c0), sa).wait()

        @pl.when(k < _GNCH // 2 - 1)
        def _():
            start_gather(c0 + 2, buf_a, ga)

        return carry

    lax.fori_loop(0, _GNCH // 2, pair, 0)
    pltpu.make_async_copy(buf_b, x_dst(_GNCH - 1), sb).wait()


def _gather_sc(emb, idx):
    return pl.kernel(
        _gather_body,
        out_type=jax.ShapeDtypeStruct((L, D), jnp.float32),
        mesh=_sc_mesh(),
        scratch_types=[
            pltpu.VMEM((_WR,), jnp.int32),
            pltpu.VMEM((_GCH, D), jnp.float32),
            pltpu.VMEM((_GCH, D), jnp.float32),
            pltpu.SemaphoreType.DMA,
            pltpu.SemaphoreType.DMA,
            pltpu.SemaphoreType.DMA,
            pltpu.SemaphoreType.DMA,
        ],
    )(emb, idx)


def _scatter_body(z_hbm, idx_hbm, s_hbm,
                  idx_v, buf_a, buf_b, acc, ia, la, lb, wa, wb):
    c = lax.axis_index("c")
    s = lax.axis_index("s")
    w = s * 2 + c
    base = pl.multiple_of(w * _WR, _WR)
    nbase = s * _NODES_PER_TILE

    zvec = jnp.zeros((16,), jnp.float32)
    for row in range(40):
        for q in range(D // 16):
            buf_a[row, pl.ds(16 * q, 16)] = zvec
    for q in range(_NODES_PER_TILE // 40):
        pltpu.sync_copy(buf_a.at[pl.ds(0, 40)],
                        acc.at[pl.ds(nbase + q * 40, 40)])
    plsc.subcore_barrier()

    def idx_src(j):
        return idx_hbm.at[pl.ds(base + j * _SCH, _SCH)]

    def fire(j, carry):
        pltpu.async_copy(idx_src(j), idx_v.at[j], ia)
        return carry
    lax.fori_loop(0, _SNCH, fire, 0)

    def drain(j, carry):
        pltpu.make_async_copy(idx_src(j), idx_v.at[j], ia).wait()
        return carry
    lax.fori_loop(0, _SNCH, drain, 0)

    def z_src(j):
        return z_hbm.at[pl.ds(base + j * _SCH, _SCH)]

    def start_scatter(j, buf, sem):
        pltpu.async_copy(buf, acc.at[idx_v.at[j]], sem, add=True)

    def wait_scatter(j, buf, sem):
        pltpu.make_async_copy(buf, acc.at[idx_v.at[j]], sem).wait()

    pltpu.async_copy(z_src(0), buf_a, la)

    def pair(k, carry2):
        j0 = 2 * k
        j1 = j0 + 1
        pltpu.make_async_copy(z_src(j0), buf_a, la).wait()
        start_scatter(j0, buf_a, wa)

        @pl.when(k > 0)
        def _():
            wait_scatter(j0 - 1, buf_b, wb)

        pltpu.async_copy(z_src(j1), buf_b, lb)
        pltpu.make_async_copy(z_src(j1), buf_b, lb).wait()
        start_scatter(j1, buf_b, wb)
        wait_scatter(j0, buf_a, wa)

        @pl.when(k < _SNCH // 2 - 1)
        def _():
            pltpu.async_copy(z_src(j0 + 2), buf_a, la)

        return carry2

    lax.fori_loop(0, _SNCH // 2, pair, 0)

    tail = _SNCH - 1
    pltpu.async_copy(z_src(tail), buf_a, la)
    pltpu.make_async_copy(z_src(tail), buf_a, la).wait()
    start_scatter(tail, buf_a, wa)
    wait_scatter(tail - 1, buf_b, wb)
    wait_scatter(tail, buf_a, wa)

    plsc.subcore_barrier()
    pltpu.sync_copy(acc.at[pl.ds(nbase, _NODES_PER_TILE)],
                    s_hbm.at[c, pl.ds(nbase, _NODES_PER_TILE)])


def _scatter_sc(z, idx):
    return pl.kernel(
        _scatter_body,
        out_type=jax.ShapeDtypeStruct((2, N_PAD, D), jnp.float32),
        mesh=_sc_mesh(),
        scratch_types=[
            pltpu.VMEM((_SNCH, _SCH), jnp.int32),
            pltpu.VMEM((_SCH, D), jnp.float32),
            pltpu.VMEM((_SCH, D), jnp.float32),
            pltpu.VMEM_SHARED((N_PAD, D), jnp.float32),
            pltpu.SemaphoreType.DMA,
            pltpu.SemaphoreType.DMA,
            pltpu.SemaphoreType.DMA,
            pltpu.SemaphoreType.DMA,
            pltpu.SemaphoreType.DMA,
        ],
    )(z, idx)


def _mish(t):
    w = jnp.exp(jnp.minimum(t, 30.0))
    n = w * (w + 2.0)
    return t * (n / (n + 2.0))


def _rel_body(x_ref, wrt_ref, br_ref, wot_ref, bo_ref, z_ref):
    x = x_ref[...].reshape(_REL_BLK, 2 * D)
    t = jnp.dot(x.astype(jnp.bfloat16), wrt_ref[...],
                preferred_element_type=jnp.float32) + br_ref[...]
    h = x + _mish(t)
    e = jnp.dot(h.astype(jnp.bfloat16), wot_ref[...],
                preferred_element_type=jnp.float32) + bo_ref[...]
    z_ref[...] = jnp.exp(e).reshape(2 * _REL_BLK, D)


def _relation_z(x, WrT, br2, WoT, bo2):
    grid = T // _REL_BLK
    return pl.pallas_call(
        _rel_body,
        grid=(grid,),
        in_specs=[
            pl.BlockSpec((2 * _REL_BLK, D), lambda i: (i, 0)),
            pl.BlockSpec((2 * D, 2 * D), lambda i: (0, 0)),
            pl.BlockSpec((1, 2 * D), lambda i: (0, 0)),
            pl.BlockSpec((2 * D, 2 * D), lambda i: (0, 0)),
            pl.BlockSpec((1, 2 * D), lambda i: (0, 0)),
        ],
        out_specs=pl.BlockSpec((2 * _REL_BLK, D), lambda i: (i, 0)),
        out_shape=jax.ShapeDtypeStruct((L, D), jnp.float32),
    )(x, WrT, br2, WoT, bo2)


def _final_body(s0_ref, s1_ref, emb_ref, wrut_ref, bru_ref, wout_ref,
                bou_ref, o_ref):
    sacc = (s0_ref[0] + s0_ref[1]) + (s1_ref[0] + s1_ref[1])
    emb = emb_ref[...]
    max_msg = jnp.where(sacc == 0.0, LOG_EPS,
                        (jnp.log(sacc) + K_SHIFT) * (1.0 / 12.0))
    x = jnp.concatenate([max_msg, emb], axis=1)
    t = jnp.dot(x, wrut_ref[...], preferred_element_type=jnp.float32) + bru_ref[...]
    h = x + _mish(t)
    o_ref[...] = jnp.dot(h, wout_ref[...], preferred_element_type=jnp.float32) + bou_ref[...]


def _final_update(s0, s1, emb, WruT, bru2, WouT, bou2):
    grid = N // _FIN_BLK
    return pl.pallas_call(
        _final_body,
        grid=(grid,),
        in_specs=[
            pl.BlockSpec((2, _FIN_BLK, D), lambda i: (0, i, 0)),
            pl.BlockSpec((2, _FIN_BLK, D), lambda i: (0, i, 0)),
            pl.BlockSpec((_FIN_BLK, D), lambda i: (i, 0)),
            pl.BlockSpec((2 * D, 2 * D), lambda i: (0, 0)),
            pl.BlockSpec((1, 2 * D), lambda i: (0, 0)),
            pl.BlockSpec((2 * D, D), lambda i: (0, 0)),
            pl.BlockSpec((1, D), lambda i: (0, 0)),
        ],
        out_specs=pl.BlockSpec((_FIN_BLK, D), lambda i: (i, 0)),
        out_shape=jax.ShapeDtypeStruct((N, D), jnp.float32),
    )(s0, s1, emb, WruT, bru2, WouT, bou2)


def kernel(object_embeddings, rel0_values, rel1_values,
           Wr0, br0, Wo0, bo0, Wr1, br1, Wo1, bo1,
           Wru, bru, Wou, bou):
    emb = object_embeddings

    bf = jnp.bfloat16
    x0 = _gather_sc(emb, rel0_values)
    z0 = _relation_z(x0, Wr0.T.astype(bf), br0.reshape(1, -1),
                     (12.0 * Wo0.T).astype(bf),
                     (12.0 * bo0 - K_SHIFT).reshape(1, -1))
    s0 = _scatter_sc(z0, rel0_values)

    x1 = _gather_sc(emb, rel1_values)
    z1 = _relation_z(x1, Wr1.T.astype(bf), br1.reshape(1, -1),
                     (12.0 * Wo1.T).astype(bf),
                     (12.0 * bo1 - K_SHIFT).reshape(1, -1))
    s1 = _scatter_sc(z1, rel1_values)

    return _final_update(s0, s1, emb, Wru.T, bru.reshape(1, -1),
                         Wou.T, bou.reshape(1, -1))
